# baseline (speedup 1.0000x reference)
import functools

import jax
import jax.numpy as jnp
from jax import lax
from jax.experimental import pallas as pl
from jax.experimental.pallas import tpu as pltpu
from jax.experimental.pallas import tpu_sc as plsc

VOCAB = 100000
WIDTH = 128
CONTEXT = 200
BATCH = 4096

ROWS = CONTEXT * BATCH
NUM_WORKERS = 32
ROWS_PER_W = ROWS // NUM_WORKERS
GROWS = 128
CHUNK = 256
GPC = CHUNK // GROWS
NCHUNK = ROWS_PER_W // CHUNK
NIDX = ROWS_PER_W // GROWS
VREGS_PER_ROW = WIDTH // 16


def _build_kernel():
    mesh = plsc.VectorSubcoreMesh(core_axis_name="c", subcore_axis_name="s")

    @functools.partial(
        pl.kernel,
        mesh=mesh,
        out_type=jax.ShapeDtypeStruct((ROWS, WIDTH), jnp.float32),
        scratch_types=[
            pltpu.VMEM((NIDX, GROWS), jnp.int32),
            pltpu.VMEM((CONTEXT, WIDTH), jnp.float32),
            pltpu.VMEM((CHUNK, WIDTH), jnp.float32),
            pltpu.VMEM((CHUNK, WIDTH), jnp.float32),
            pltpu.SemaphoreType.DMA,
            pltpu.SemaphoreType.DMA,
            pltpu.SemaphoreType.DMA,
            pltpu.SemaphoreType.DMA,
            pltpu.SemaphoreType.DMA,
            pltpu.SemaphoreType.DMA,
        ],
    )
    def body(idx_hbm, table_hbm, pos_hbm, out_hbm, idx_v, pos_v,
             buf0, buf1, gsem00, gsem01, gsem10, gsem11, ssem0, ssem1):
        buf = (buf0, buf1)
        gsem = ((gsem00, gsem01), (gsem10, gsem11))
        ssem = (ssem0, ssem1)

        wid = lax.axis_index("c") * 16 + lax.axis_index("s")
        base_row = wid * ROWS_PER_W
        pltpu.sync_copy(idx_hbm.at[pl.ds(wid * NIDX, NIDX)], idx_v)
        pltpu.sync_copy(pos_hbm, pos_v)

        def fire_gather(g, b, h):
            return pltpu.async_copy(
                table_hbm.at[idx_v.at[g * GPC + h]],
                buf[b].at[pl.ds(h * GROWS, GROWS)], gsem[b][h])

        def add_half(g, b, h):
            l = (base_row + g * CHUNK) // BATCH
            pks = [pos_v[l, pl.ds(16 * k, 16)] for k in range(VREGS_PER_ROW)]
            bb = buf[b]

            def add_row(j, c):
                for k in range(VREGS_PER_ROW):
                    plsc.addupdate(bb.at[j, pl.ds(16 * k, 16)], pks[k])
                return c

            lax.fori_loop(h * GROWS, (h + 1) * GROWS, add_row, 0, unroll=4)

        def fire_store(g, b):
            row0 = base_row + g * CHUNK
            pltpu.async_copy(buf[b], out_hbm.at[pl.ds(row0, CHUNK)], ssem[b])

        def wait_store(g, b):
            row0 = base_row + g * CHUNK
            pltpu.make_async_copy(buf[b], out_hbm.at[pl.ds(row0, CHUNK)],
                                  ssem[b]).wait()

        def chunk_pair(g0, first):
            g1 = g0 + 1
            if not first:
                wait_store(g0 - 2, 0)
            hA0 = fire_gather(g0, 0, 0)
            hA1 = fire_gather(g0, 0, 1)
            hA0.wait()
            add_half(g0, 0, 0)
            if not first:
                wait_store(g1 - 2, 1)
            hB0 = fire_gather(g1, 1, 0)
            hA1.wait()
            add_half(g0, 0, 1)
            fire_store(g0, 0)
            hB1 = fire_gather(g1, 1, 1)
            hB0.wait()
            add_half(g1, 1, 0)
            hB1.wait()
            add_half(g1, 1, 1)
            fire_store(g1, 1)

        chunk_pair(0, True)

        def outer(go, carry):
            chunk_pair(go * 2, False)
            return carry

        lax.fori_loop(1, NCHUNK // 2, outer, 0)

        for g in (NCHUNK - 2, NCHUNK - 1):
            wait_store(g, g % 2)

    return body


_sc_kernel = _build_kernel()


def kernel(text, token_embedding, positional_embedding):
    idx = jnp.transpose(text).astype(jnp.int32).reshape(ROWS // GROWS, GROWS)
    out = _sc_kernel(idx, token_embedding, positional_embedding)
    return out.reshape(CONTEXT, BATCH, WIDTH)


# speedup vs baseline: 1.0053x; 1.0053x over previous
import functools

import jax
import jax.numpy as jnp
from jax import lax
from jax.experimental import pallas as pl
from jax.experimental.pallas import tpu as pltpu
from jax.experimental.pallas import tpu_sc as plsc

VOCAB = 100000
WIDTH = 128
CONTEXT = 200
BATCH = 4096

ROWS = CONTEXT * BATCH
NUM_WORKERS = 32
ROWS_PER_W = ROWS // NUM_WORKERS
GROWS = 128
CHUNK = 256
GPC = CHUNK // GROWS
NCHUNK = ROWS_PER_W // CHUNK
NIDX = ROWS_PER_W // GROWS
VREGS_PER_ROW = WIDTH // 16


def _build_kernel():
    mesh = plsc.VectorSubcoreMesh(core_axis_name="c", subcore_axis_name="s")

    @functools.partial(
        pl.kernel,
        mesh=mesh,
        out_type=jax.ShapeDtypeStruct((ROWS, WIDTH), jnp.float32),
        scratch_types=[
            pltpu.VMEM((NIDX, GROWS), jnp.int32),
            pltpu.VMEM((CONTEXT, WIDTH), jnp.float32),
            pltpu.VMEM((CHUNK, WIDTH), jnp.float32),
            pltpu.VMEM((CHUNK, WIDTH), jnp.float32),
            pltpu.SemaphoreType.DMA,
            pltpu.SemaphoreType.DMA,
            pltpu.SemaphoreType.DMA,
            pltpu.SemaphoreType.DMA,
            pltpu.SemaphoreType.DMA,
            pltpu.SemaphoreType.DMA,
        ],
    )
    def body(idx_hbm, table_hbm, pos_hbm, out_hbm, idx_v, pos_v,
             buf0, buf1, gsem00, gsem01, gsem10, gsem11, ssem0, ssem1):
        buf = (buf0, buf1)
        gsem = ((gsem00, gsem01), (gsem10, gsem11))
        ssem = (ssem0, ssem1)

        wid = lax.axis_index("s") * 2 + lax.axis_index("c")
        base_row = wid * ROWS_PER_W
        pltpu.sync_copy(idx_hbm.at[pl.ds(wid * NIDX, NIDX)], idx_v)
        pltpu.sync_copy(pos_hbm, pos_v)

        def fire_gather(g, b, h):
            return pltpu.async_copy(
                table_hbm.at[idx_v.at[g * GPC + h]],
                buf[b].at[pl.ds(h * GROWS, GROWS)], gsem[b][h])

        def add_half(g, b, h):
            l = (base_row + g * CHUNK) // BATCH
            pks = [pos_v[l, pl.ds(16 * k, 16)] for k in range(VREGS_PER_ROW)]
            bb = buf[b]

            def add_row(j, c):
                for k in range(VREGS_PER_ROW):
                    plsc.addupdate(bb.at[j, pl.ds(16 * k, 16)], pks[k])
                return c

            lax.fori_loop(h * GROWS, (h + 1) * GROWS, add_row, 0, unroll=4)

        def fire_store(g, b):
            row0 = base_row + g * CHUNK
            pltpu.async_copy(buf[b], out_hbm.at[pl.ds(row0, CHUNK)], ssem[b])

        def wait_store(g, b):
            row0 = base_row + g * CHUNK
            pltpu.make_async_copy(buf[b], out_hbm.at[pl.ds(row0, CHUNK)],
                                  ssem[b]).wait()

        def chunk_pair(g0, first):
            g1 = g0 + 1
            if not first:
                wait_store(g0 - 2, 0)
            hA0 = fire_gather(g0, 0, 0)
            hA1 = fire_gather(g0, 0, 1)
            hA0.wait()
            add_half(g0, 0, 0)
            if not first:
                wait_store(g1 - 2, 1)
            hB0 = fire_gather(g1, 1, 0)
            hA1.wait()
            add_half(g0, 0, 1)
            fire_store(g0, 0)
            hB1 = fire_gather(g1, 1, 1)
            hB0.wait()
            add_half(g1, 1, 0)
            hB1.wait()
            add_half(g1, 1, 1)
            fire_store(g1, 1)

        chunk_pair(0, True)

        def outer(go, carry):
            chunk_pair(go * 2, False)
            return carry

        lax.fori_loop(1, NCHUNK // 2, outer, 0)

        for g in (NCHUNK - 2, NCHUNK - 1):
            wait_store(g, g % 2)

    return body


_sc_kernel = _build_kernel()


def kernel(text, token_embedding, positional_embedding):
    idx = jnp.transpose(text).astype(jnp.int32).reshape(ROWS // GROWS, GROWS)
    out = _sc_kernel(idx, token_embedding, positional_embedding)
    return out.reshape(CONTEXT, BATCH, WIDTH)
